# Initial kernel scaffold; baseline (speedup 1.0000x reference)
import jax
import jax.numpy as jnp
from jax.experimental import pallas as pl


def kernel(centerness, bbox_pred, cls_scores, points):
    raise NotImplementedError("write your pallas kernel here")


# speedup vs baseline: 6.6689x; 6.6689x over previous
import functools

import jax
import jax.numpy as jnp
from jax.experimental import pallas as pl
from jax.experimental.pallas import tpu as pltpu

N = 20000
C = 18
NMS_PRE = 1000
IOU_THR = 0.5

LANES = 128
ROWS_IN = 157
NP = ROWS_IN * LANES
ROWS_SORT = 256
SN = ROWS_SORT * LANES
K = 1024
PAYW = 16
NPAD = 20480
CHUNK = 2048


def _roll(x, shift, axis):
    return pltpu.roll(x, shift % x.shape[axis], axis)


def _topk_kernel(cls_ref, cen_ref, kidx_ref, kscore_ref, lbl_ref):
    cls3 = cls_ref[...]
    cen = cen_ref[...]
    s = jax.nn.sigmoid(cls3) * jax.nn.sigmoid(cen)[..., None]
    m_raw = jnp.max(s, axis=-1)
    ciota = jax.lax.broadcasted_iota(jnp.int32, s.shape, 2)
    lbl = jnp.min(jnp.where(s == m_raw[..., None], ciota, 10 ** 6), axis=-1)
    lbl_ref[...] = lbl.astype(jnp.float32)

    r_i = jax.lax.broadcasted_iota(jnp.int32, (ROWS_IN, LANES), 0)
    c_i = jax.lax.broadcasted_iota(jnp.int32, (ROWS_IN, LANES), 1)
    flat = r_i * LANES + c_i
    m = jnp.where(flat < N, m_raw, -1.0)

    val = jnp.concatenate([m, jnp.full((ROWS_SORT - ROWS_IN, LANES), -1.0,
                                       jnp.float32)], axis=0)
    r_s = jax.lax.broadcasted_iota(jnp.int32, (ROWS_SORT, LANES), 0)
    c_s = jax.lax.broadcasted_iota(jnp.int32, (ROWS_SORT, LANES), 1)
    idx = r_s * LANES + c_s

    for t in range(1, 16):
        k = 1 << t
        j = k >> 1
        while j >= 1:
            if j < LANES:
                bit = (c_s & j) != 0
                pv = jnp.where(bit, _roll(val, j, 1), _roll(val, -j, 1))
                px = jnp.where(bit, _roll(idx, j, 1), _roll(idx, -j, 1))
            else:
                jr = j // LANES
                bit = (r_s & jr) != 0
                pv = jnp.where(bit, _roll(val, jr, 0), _roll(val, -jr, 0))
                px = jnp.where(bit, _roll(idx, jr, 0), _roll(idx, -jr, 0))
            self_first = (val > pv) | ((val == pv) & (idx < px))
            if k < SN:
                if k < LANES:
                    desc = (c_s & k) == 0
                else:
                    desc = (r_s & (k // LANES)) == 0
                lower = ~bit
                keep_self = self_first ^ (desc ^ lower)
            else:
                keep_self = self_first ^ bit
            val = jnp.where(keep_self, val, pv)
            idx = jnp.where(keep_self, idx, px)
            j >>= 1

    kidx_ref[...] = idx[:K // LANES, :]
    kscore_ref[...] = val[:K // LANES, :]


def _nms_kernel(kidx_col_ref, kidx_row_ref, kscore_row_ref,
                pcol_ref, prow_ref,
                boxes_ref, scores_ref, labels_ref, s_ref):
    kidx_col = kidx_col_ref[...]
    kidx_row = kidx_row_ref[...]

    gcol = jnp.zeros((K, PAYW), jnp.float32)
    for ci in range(NPAD // CHUNK):
        it = jax.lax.broadcasted_iota(jnp.int32, (1, CHUNK), 1) + ci * CHUNK
        onehot = (kidx_col == it).astype(jnp.float32)
        gcol = gcol + jnp.dot(onehot, pcol_ref[ci * CHUNK:(ci + 1) * CHUNK, :],
                              preferred_element_type=jnp.float32)
    grow = jnp.zeros((PAYW, K), jnp.float32)
    for ci in range(NPAD // CHUNK):
        it = jax.lax.broadcasted_iota(jnp.int32, (CHUNK, 1), 0) + ci * CHUNK
        onehot = (it == kidx_row).astype(jnp.float32)
        grow = grow + jnp.dot(prow_ref[:, ci * CHUNK:(ci + 1) * CHUNK], onehot,
                              preferred_element_type=jnp.float32)

    dist_c = jnp.exp(gcol[:, 0:6])
    lo_c = gcol[:, 6:9] - dist_c[:, 0:3]
    hi_c = gcol[:, 6:9] + dist_c[:, 3:6]
    dist_r = jnp.exp(grow[0:6, :])
    lo_r = grow[6:9, :] - dist_r[0:3, :]
    hi_r = grow[6:9, :] + dist_r[3:6, :]

    whd_c = jnp.clip(hi_c - lo_c, 0.0)
    vol_c = whd_c[:, 0:1] * whd_c[:, 1:2] * whd_c[:, 2:3]
    whd_r = jnp.clip(hi_r - lo_r, 0.0)
    vol_r = whd_r[0:1, :] * whd_r[1:2, :] * whd_r[2:3, :]

    boxes_ref[...] = jnp.concatenate([lo_c, hi_c], axis=1)

    TR = 128
    for rt in range(K // TR):
        sl = slice(rt * TR, (rt + 1) * TR)
        inter = jnp.ones((TR, K), jnp.float32)
        for d in range(3):
            lt = jnp.maximum(lo_c[sl, d:d + 1], lo_r[d:d + 1, :])
            rb = jnp.minimum(hi_c[sl, d:d + 1], hi_r[d:d + 1, :])
            inter = inter * jnp.clip(rb - lt, 0.0)
        union = vol_c[sl] + vol_r - inter
        iou = inter / jnp.maximum(union, 1e-6)
        s_ref[sl, :] = (iou > IOU_THR).astype(jnp.float32)

    liota = jax.lax.broadcasted_iota(jnp.int32, (1, K), 1)

    def body(i, keep):
        srow = s_ref[pl.ds(i, 1), :]
        ki = jnp.sum(jnp.where(liota == i, keep, 0.0))
        sup = srow * ki * (liota > i).astype(jnp.float32)
        return keep * (1.0 - sup)

    keep = jax.lax.fori_loop(0, NMS_PRE, body,
                             jnp.ones((1, K), jnp.float32))
    scores_ref[...] = kscore_row_ref[...] * keep
    labels_ref[...] = grow[9:10, :]


@jax.jit
def kernel(centerness, bbox_pred, cls_scores, points):
    cls3 = jnp.pad(cls_scores, ((0, NP - N), (0, 0))).reshape(ROWS_IN, LANES, C)
    cen = jnp.pad(centerness[:, 0], (0, NP - N)).reshape(ROWS_IN, LANES)

    kidx, kscore, lbl = pl.pallas_call(
        _topk_kernel,
        out_shape=[
            jax.ShapeDtypeStruct((K // LANES, LANES), jnp.int32),
            jax.ShapeDtypeStruct((K // LANES, LANES), jnp.float32),
            jax.ShapeDtypeStruct((ROWS_IN, LANES), jnp.float32),
        ],
    )(cls3, cen)

    labels_all = lbl.reshape(NP)[:N]
    pay = jnp.concatenate(
        [bbox_pred, points, labels_all[:, None],
         jnp.zeros((N, PAYW - 10), jnp.float32)], axis=1)
    pcol = jnp.pad(pay, ((0, NPAD - N), (0, 0)))
    prow = pcol.T
    kflat = kidx.reshape(K)
    sflat = kscore.reshape(K)

    boxes, scores, labels = pl.pallas_call(
        _nms_kernel,
        out_shape=[
            jax.ShapeDtypeStruct((K, 6), jnp.float32),
            jax.ShapeDtypeStruct((1, K), jnp.float32),
            jax.ShapeDtypeStruct((1, K), jnp.float32),
        ],
        scratch_shapes=[pltpu.VMEM((K, K), jnp.float32)],
    )(kflat[:, None], kflat[None, :], sflat[None, :], pcol, prow)

    return (boxes[:NMS_PRE], scores[0, :NMS_PRE],
            labels[0, :NMS_PRE].astype(jnp.int32))
